# baseline (speedup 1.0000x reference)
import functools

import jax
import jax.numpy as jnp
from jax import lax
from jax.experimental import pallas as pl
from jax.experimental.pallas import tpu as pltpu
from jax.experimental.pallas import tpu_sc as plsc

N_NODES = 10000
N_EDGES = 320000
D = 128

NC = 2
NS = 16
NW = NC * NS
K = 128
CH = 80
CPH = 40
CA = 160
CB = 0
CPG = 16
EPW = CH * K
EPAD = NW * EPW
TRASH = N_NODES
NP = 10240
RPT = NP // NS
DEGW = 16

_mesh = plsc.VectorSubcoreMesh(core_axis_name="c", subcore_axis_name="s")


def _zero_rows(buf, nrows, ncolchunks):
    z = jnp.zeros((16,), jnp.float32)

    def body(i, _):
        for j in range(ncolchunks):
            buf[i, pl.ds(j * 16, 16)] = z
        return 0

    lax.fori_loop(0, nrows, body, 0)


@functools.partial(
    pl.kernel,
    out_type=jax.ShapeDtypeStruct((NC, NP, DEGW), jnp.float32),
    mesh=_mesh,
    scratch_types=[
        pltpu.VMEM_SHARED((NP, DEGW), jnp.float32),
        pltpu.VMEM((CH, K), jnp.int32),
        pltpu.VMEM((K, DEGW), jnp.float32),
    ],
)
def _deg_kernel(dst_hbm, out_hbm, acc, db, rows):
    c = lax.axis_index("c")
    s = lax.axis_index("s")
    wid = s * NC + c
    _zero_rows(rows, K, DEGW // 16)
    for k in range(RPT // K):
        pltpu.sync_copy(rows, acc.at[pl.ds(s * RPT + k * K, K)])
    one = jnp.full((16,), 1.0, jnp.float32)

    def fill(i, _):
        rows[i, pl.ds(0, 16)] = one
        return 0

    lax.fori_loop(0, K, fill, 0)
    pltpu.sync_copy(dst_hbm.at[pl.ds(wid * CH, CH)], db)
    plsc.subcore_barrier()

    def chunk(j, _):
        pltpu.sync_copy(rows, acc.at[db.at[j]], add=True)
        return 0

    lax.fori_loop(0, CH, chunk, 0)
    plsc.subcore_barrier()
    pltpu.sync_copy(
        acc.at[pl.ds(s * RPT, RPT)], out_hbm.at[c].at[pl.ds(s * RPT, RPT)]
    )


@functools.partial(
    pl.kernel,
    out_type=jax.ShapeDtypeStruct((NC, NP, D), jnp.float32),
    mesh=_mesh,
    scratch_types=[
        pltpu.VMEM_SHARED((NP, D), jnp.float32),
        pltpu.VMEM((CPG, K), jnp.int32),
        pltpu.VMEM((CPG, K), jnp.int32),
        pltpu.VMEM((K, D), jnp.float32),
        pltpu.VMEM((K, D), jnp.float32),
        pltpu.SemaphoreType.DMA,
        pltpu.SemaphoreType.DMA,
    ],
)
def _scatter_kernel(g_hbm, src_hbm, dst_hbm, out_hbm, acc, sb, db, ra, rb, sa, sb_sem):
    c = lax.axis_index("c")
    s = lax.axis_index("s")
    _zero_rows(ra, K, D // 16)
    for k in range(RPT // K):
        pltpu.sync_copy(ra, acc.at[pl.ds(s * RPT + k * K, K)])
    plsc.subcore_barrier()

    my_groups = jnp.where(c == 0, CA // CPG, CB // CPG)
    tile_base = jnp.where(c == 0, s * CA, NS * CA + s * CB)

    for gi in range(CA // CPG):

        @pl.when(gi < my_groups)
        def _():
            base = tile_base + gi * CPG
            pltpu.sync_copy(src_hbm.at[pl.ds(base, CPG)], sb)
            pltpu.sync_copy(dst_hbm.at[pl.ds(base, CPG)], db)
            pltpu.async_copy(g_hbm.at[sb.at[0]], ra, sa)

            def chunk(m, _):
                j = 2 * m
                pltpu.make_async_copy(g_hbm.at[sb.at[j]], ra, sa).wait()
                pltpu.async_copy(g_hbm.at[sb.at[j + 1]], rb, sa)
                pltpu.sync_copy(ra, acc.at[db.at[j]], add=True)
                pltpu.make_async_copy(g_hbm.at[sb.at[j + 1]], rb, sa).wait()
                pltpu.async_copy(g_hbm.at[sb.at[j + 2]], ra, sa)
                pltpu.sync_copy(rb, acc.at[db.at[j + 1]], add=True)
                return 0

            lax.fori_loop(0, CPG // 2 - 1, chunk, 0)
            pltpu.make_async_copy(g_hbm.at[sb.at[CPG - 2]], ra, sa).wait()
            pltpu.async_copy(g_hbm.at[sb.at[CPG - 1]], rb, sa)
            pltpu.sync_copy(ra, acc.at[db.at[CPG - 2]], add=True)
            pltpu.make_async_copy(g_hbm.at[sb.at[CPG - 1]], rb, sa).wait()
            pltpu.sync_copy(rb, acc.at[db.at[CPG - 1]], add=True)

    plsc.subcore_barrier()
    pltpu.sync_copy(
        acc.at[pl.ds(s * RPT, RPT)], out_hbm.at[c].at[pl.ds(s * RPT, RPT)]
    )


def _dinv_of(deg_ref):
    d = deg_ref[0, :, 0] + deg_ref[1, :, 0] + 1.0
    return lax.rsqrt(d)[:, None]


def _tc1_body(deg_ref, x_ref, w_ref, o_ref):
    h = jnp.dot(x_ref[...], w_ref[...], preferred_element_type=jnp.float32)
    o_ref[...] = h * _dinv_of(deg_ref)


def _tc2_body(deg_ref, a_ref, g_ref, b_ref, w_ref, o_ref):
    dinv = _dinv_of(deg_ref)
    t = (a_ref[0] + a_ref[1] + g_ref[...]) * dinv + b_ref[...]
    t = jnp.maximum(t, 0.0)
    o_ref[...] = jnp.dot(t, w_ref[...], preferred_element_type=jnp.float32) * dinv


def _tc3_body(deg_ref, a_ref, g_ref, b_ref, o_ref):
    dinv = _dinv_of(deg_ref)
    o_ref[...] = (a_ref[0] + a_ref[1] + g_ref[...]) * dinv + b_ref[...]


_BT = 1024


def _tc1(degacc, x_pad, W1):
    grid = (NP // _BT,)
    return pl.pallas_call(
        _tc1_body,
        grid=grid,
        in_specs=[
            pl.BlockSpec((NC, _BT, DEGW), lambda i: (0, i, 0)),
            pl.BlockSpec((_BT, D), lambda i: (i, 0)),
            pl.BlockSpec((D, D), lambda i: (0, 0)),
        ],
        out_specs=pl.BlockSpec((_BT, D), lambda i: (i, 0)),
        out_shape=jax.ShapeDtypeStruct((NP, D), jnp.float32),
    )(degacc, x_pad, W1)


def _tc2(degacc, acc1, g1, b1, W2):
    grid = (NP // _BT,)
    return pl.pallas_call(
        _tc2_body,
        grid=grid,
        in_specs=[
            pl.BlockSpec((NC, _BT, DEGW), lambda i: (0, i, 0)),
            pl.BlockSpec((NC, _BT, D), lambda i: (0, i, 0)),
            pl.BlockSpec((_BT, D), lambda i: (i, 0)),
            pl.BlockSpec((1, D), lambda i: (0, 0)),
            pl.BlockSpec((D, D), lambda i: (0, 0)),
        ],
        out_specs=pl.BlockSpec((_BT, D), lambda i: (i, 0)),
        out_shape=jax.ShapeDtypeStruct((NP, D), jnp.float32),
    )(degacc, acc1, g1, b1, W2)


def _tc3(degacc, acc2, g2, b2):
    bt = 1000
    grid = (N_NODES // bt,)
    return pl.pallas_call(
        _tc3_body,
        grid=grid,
        in_specs=[
            pl.BlockSpec((NC, bt, DEGW), lambda i: (0, i, 0)),
            pl.BlockSpec((NC, bt, D), lambda i: (0, i, 0)),
            pl.BlockSpec((bt, D), lambda i: (i, 0)),
            pl.BlockSpec((1, D), lambda i: (0, 0)),
        ],
        out_specs=pl.BlockSpec((bt, D), lambda i: (i, 0)),
        out_shape=jax.ShapeDtypeStruct((N_NODES, D), jnp.float32),
    )(degacc, acc2, g2, b2)


def kernel(x, edge_index, W1, b1, W2, b2):
    src = edge_index[0].astype(jnp.int32)
    dst = edge_index[1].astype(jnp.int32)
    pad = EPAD - N_EDGES
    fillv = jnp.full((pad,), TRASH, jnp.int32)
    src = jnp.concatenate([src, fillv]).reshape(NW * CH, K)
    dst = jnp.concatenate([dst, fillv]).reshape(NW * CH, K)
    x_pad = jnp.pad(x, ((0, NP - N_NODES), (0, 0)))

    degacc = _deg_kernel(dst)
    g1 = _tc1(degacc, x_pad, W1)
    acc1 = _scatter_kernel(g1, src, dst)
    g2 = _tc2(degacc, acc1, g1, b1.reshape(1, D), W2)
    acc2 = _scatter_kernel(g2, src, dst)
    return _tc3(degacc, acc2, g2, b2.reshape(1, D))


# speedup vs baseline: 1.3497x; 1.3497x over previous
import functools

import jax
import jax.numpy as jnp
from jax import lax
from jax.experimental import pallas as pl
from jax.experimental.pallas import tpu as pltpu
from jax.experimental.pallas import tpu_sc as plsc

N_NODES = 10000
N_EDGES = 320000
D = 128

NC = 2
NS = 16
NW = NC * NS
K = 128
CH = 80
CPH = 40
CA = 144
CB = 16
CPG = 16
EPW = CH * K
EPAD = NW * EPW
TRASH = N_NODES
NP = 10240
RPT = NP // NS
DEGW = 16

_mesh = plsc.VectorSubcoreMesh(core_axis_name="c", subcore_axis_name="s")


def _zero_rows(buf, nrows, ncolchunks):
    z = jnp.zeros((16,), jnp.float32)

    def body(i, _):
        for j in range(ncolchunks):
            buf[i, pl.ds(j * 16, 16)] = z
        return 0

    lax.fori_loop(0, nrows, body, 0)


@functools.partial(
    pl.kernel,
    out_type=jax.ShapeDtypeStruct((NC, NP, DEGW), jnp.float32),
    mesh=_mesh,
    scratch_types=[
        pltpu.VMEM_SHARED((NP, DEGW), jnp.float32),
        pltpu.VMEM((CH, K), jnp.int32),
        pltpu.VMEM((K, DEGW), jnp.float32),
    ],
)
def _deg_kernel(dst_hbm, out_hbm, acc, db, rows):
    c = lax.axis_index("c")
    s = lax.axis_index("s")
    wid = s * NC + c
    _zero_rows(rows, K, DEGW // 16)
    for k in range(RPT // K):
        pltpu.sync_copy(rows, acc.at[pl.ds(s * RPT + k * K, K)])
    one = jnp.full((16,), 1.0, jnp.float32)

    def fill(i, _):
        rows[i, pl.ds(0, 16)] = one
        return 0

    lax.fori_loop(0, K, fill, 0)
    pltpu.sync_copy(dst_hbm.at[pl.ds(wid * CH, CH)], db)
    plsc.subcore_barrier()

    def chunk(j, _):
        pltpu.sync_copy(rows, acc.at[db.at[j]], add=True)
        return 0

    lax.fori_loop(0, CH, chunk, 0)
    plsc.subcore_barrier()
    pltpu.sync_copy(
        acc.at[pl.ds(s * RPT, RPT)], out_hbm.at[c].at[pl.ds(s * RPT, RPT)]
    )


@functools.partial(
    pl.kernel,
    out_type=jax.ShapeDtypeStruct((NC, NP, D), jnp.float32),
    mesh=_mesh,
    scratch_types=[
        pltpu.VMEM_SHARED((NP, D), jnp.float32),
        pltpu.VMEM((CPG, K), jnp.int32),
        pltpu.VMEM((CPG, K), jnp.int32),
        pltpu.VMEM((K, D), jnp.float32),
        pltpu.VMEM((K, D), jnp.float32),
        pltpu.SemaphoreType.DMA,
        pltpu.SemaphoreType.DMA,
    ],
)
def _scatter_kernel(g_hbm, src_hbm, dst_hbm, out_hbm, acc, sb, db, ra, rb, sa, sb_sem):
    c = lax.axis_index("c")
    s = lax.axis_index("s")
    _zero_rows(ra, K, D // 16)
    for k in range(RPT // K):
        pltpu.sync_copy(ra, acc.at[pl.ds(s * RPT + k * K, K)])
    plsc.subcore_barrier()

    my_groups = jnp.where(c == 0, CA // CPG, CB // CPG)
    tile_base = jnp.where(c == 0, s * CA, NS * CA + s * CB)

    for gi in range(CA // CPG):

        @pl.when(gi < my_groups)
        def _():
            base = tile_base + gi * CPG
            pltpu.sync_copy(src_hbm.at[pl.ds(base, CPG)], sb)
            pltpu.sync_copy(dst_hbm.at[pl.ds(base, CPG)], db)
            pltpu.async_copy(g_hbm.at[sb.at[0]], ra, sa)

            def chunk(m, _):
                j = 2 * m
                pltpu.make_async_copy(g_hbm.at[sb.at[j]], ra, sa).wait()
                pltpu.async_copy(g_hbm.at[sb.at[j + 1]], rb, sa)
                pltpu.sync_copy(ra, acc.at[db.at[j]], add=True)
                pltpu.make_async_copy(g_hbm.at[sb.at[j + 1]], rb, sa).wait()
                pltpu.async_copy(g_hbm.at[sb.at[j + 2]], ra, sa)
                pltpu.sync_copy(rb, acc.at[db.at[j + 1]], add=True)
                return 0

            lax.fori_loop(0, CPG // 2 - 1, chunk, 0)
            pltpu.make_async_copy(g_hbm.at[sb.at[CPG - 2]], ra, sa).wait()
            pltpu.async_copy(g_hbm.at[sb.at[CPG - 1]], rb, sa)
            pltpu.sync_copy(ra, acc.at[db.at[CPG - 2]], add=True)
            pltpu.make_async_copy(g_hbm.at[sb.at[CPG - 1]], rb, sa).wait()
            pltpu.sync_copy(rb, acc.at[db.at[CPG - 1]], add=True)

    plsc.subcore_barrier()
    pltpu.sync_copy(
        acc.at[pl.ds(s * RPT, RPT)], out_hbm.at[c].at[pl.ds(s * RPT, RPT)]
    )


def _dinv_of(deg_ref):
    d = deg_ref[0, :, 0] + deg_ref[1, :, 0] + 1.0
    return lax.rsqrt(d)[:, None]


def _tc1_body(deg_ref, x_ref, w_ref, o_ref):
    h = jnp.dot(x_ref[...], w_ref[...], preferred_element_type=jnp.float32)
    o_ref[...] = h * _dinv_of(deg_ref)


def _tc2_body(deg_ref, a_ref, g_ref, b_ref, w_ref, o_ref):
    dinv = _dinv_of(deg_ref)
    t = (a_ref[0] + a_ref[1] + g_ref[...]) * dinv + b_ref[...]
    t = jnp.maximum(t, 0.0)
    o_ref[...] = jnp.dot(t, w_ref[...], preferred_element_type=jnp.float32) * dinv


def _tc3_body(deg_ref, a_ref, g_ref, b_ref, o_ref):
    dinv = _dinv_of(deg_ref)
    o_ref[...] = (a_ref[0] + a_ref[1] + g_ref[...]) * dinv + b_ref[...]


_BT = 1024


def _tc1(degacc, x_pad, W1):
    grid = (NP // _BT,)
    return pl.pallas_call(
        _tc1_body,
        grid=grid,
        in_specs=[
            pl.BlockSpec((NC, _BT, DEGW), lambda i: (0, i, 0)),
            pl.BlockSpec((_BT, D), lambda i: (i, 0)),
            pl.BlockSpec((D, D), lambda i: (0, 0)),
        ],
        out_specs=pl.BlockSpec((_BT, D), lambda i: (i, 0)),
        out_shape=jax.ShapeDtypeStruct((NP, D), jnp.float32),
    )(degacc, x_pad, W1)


def _tc2(degacc, acc1, g1, b1, W2):
    grid = (NP // _BT,)
    return pl.pallas_call(
        _tc2_body,
        grid=grid,
        in_specs=[
            pl.BlockSpec((NC, _BT, DEGW), lambda i: (0, i, 0)),
            pl.BlockSpec((NC, _BT, D), lambda i: (0, i, 0)),
            pl.BlockSpec((_BT, D), lambda i: (i, 0)),
            pl.BlockSpec((1, D), lambda i: (0, 0)),
            pl.BlockSpec((D, D), lambda i: (0, 0)),
        ],
        out_specs=pl.BlockSpec((_BT, D), lambda i: (i, 0)),
        out_shape=jax.ShapeDtypeStruct((NP, D), jnp.float32),
    )(degacc, acc1, g1, b1, W2)


def _tc3(degacc, acc2, g2, b2):
    bt = 1000
    grid = (N_NODES // bt,)
    return pl.pallas_call(
        _tc3_body,
        grid=grid,
        in_specs=[
            pl.BlockSpec((NC, bt, DEGW), lambda i: (0, i, 0)),
            pl.BlockSpec((NC, bt, D), lambda i: (0, i, 0)),
            pl.BlockSpec((bt, D), lambda i: (i, 0)),
            pl.BlockSpec((1, D), lambda i: (0, 0)),
        ],
        out_specs=pl.BlockSpec((bt, D), lambda i: (i, 0)),
        out_shape=jax.ShapeDtypeStruct((N_NODES, D), jnp.float32),
    )(degacc, acc2, g2, b2)


def kernel(x, edge_index, W1, b1, W2, b2):
    src = edge_index[0].astype(jnp.int32)
    dst = edge_index[1].astype(jnp.int32)
    pad = EPAD - N_EDGES
    fillv = jnp.full((pad,), TRASH, jnp.int32)
    src = jnp.concatenate([src, fillv]).reshape(NW * CH, K)
    dst = jnp.concatenate([dst, fillv]).reshape(NW * CH, K)
    x_pad = jnp.pad(x, ((0, NP - N_NODES), (0, 0)))

    degacc = _deg_kernel(dst)
    g1 = _tc1(degacc, x_pad, W1)
    acc1 = _scatter_kernel(g1, src, dst)
    g2 = _tc2(degacc, acc1, g1, b1.reshape(1, D), W2)
    acc2 = _scatter_kernel(g2, src, dst)
    return _tc3(degacc, acc2, g2, b2.reshape(1, D))
